# baseline (speedup 1.0000x reference)
import functools

import jax
import jax.numpy as jnp
from jax import lax
from jax.experimental import pallas as pl
from jax.experimental.pallas import tpu as pltpu
from jax.experimental.pallas import tpu_sc as plsc

N = 10000
D = 128
E = 320000
NC = 2
NS = 16
NW = NC * NS

C = 128
N_PAD = 10112
RPT_PAD = N_PAD // NS

E_PER_W = 10240
E_PAD = E_PER_W * NW
E_CHUNKS = E_PER_W // C

EP_PER_W = 20480
EP_PAD = EP_PER_W * NW
EP_CHUNKS = EP_PER_W // C

_MESH = plsc.VectorSubcoreMesh(core_axis_name="c", subcore_axis_name="s")


@functools.partial(
    pl.kernel,
    out_type=jax.ShapeDtypeStruct((NC, N_PAD, D), jnp.float32),
    mesh=_MESH,
    scratch_types=[
        pltpu.VMEM((EP_CHUNKS, C), jnp.int32),
        pltpu.VMEM((C, D), jnp.float32),
        pltpu.VMEM_SHARED((N_PAD, D), jnp.float32),
    ],
)
def _deg_kernel(ep_hbm, ones_hbm, zeros_hbm, out_hbm, idx_v, ones_v, acc_sh):
    c = lax.axis_index("c")
    s = lax.axis_index("s")
    w = s * NC + c
    pltpu.sync_copy(zeros_hbm.at[pl.ds(s * RPT_PAD, RPT_PAD)],
                    acc_sh.at[pl.ds(s * RPT_PAD, RPT_PAD)])
    pltpu.sync_copy(ones_hbm, ones_v)
    pltpu.sync_copy(ep_hbm.at[w], idx_v)
    plsc.subcore_barrier()

    def body(k, carry):
        pltpu.sync_copy(ones_v, acc_sh.at[idx_v.at[k]], add=True)
        return carry

    lax.fori_loop(0, EP_CHUNKS, body, 0)
    plsc.subcore_barrier()
    pltpu.sync_copy(acc_sh.at[pl.ds(s * RPT_PAD, RPT_PAD)],
                    out_hbm.at[c, pl.ds(s * RPT_PAD, RPT_PAD)])


@functools.partial(
    pl.kernel,
    out_type=jax.ShapeDtypeStruct((NC, N_PAD, D), jnp.float32),
    mesh=_MESH,
    scratch_types=[
        pltpu.VMEM((E_CHUNKS // 2, C), jnp.int32),
        pltpu.VMEM((E_CHUNKS, C), jnp.int32),
        pltpu.VMEM((C, D), jnp.float32),
        pltpu.VMEM((C, D), jnp.float32),
        pltpu.SemaphoreType.DMA,
        pltpu.SemaphoreType.DMA,
        pltpu.VMEM_SHARED((N_PAD, D), jnp.float32),
    ],
)
def _spmm_kernel(hs_hbm, src_hbm, dst_hbm, zeros_hbm, out_hbm,
                 src_m, dst_m, rows0, rows1, sem0, sem1, acc_sh):
    c = lax.axis_index("c")
    s = lax.axis_index("s")
    w = s * NC + c
    pltpu.sync_copy(zeros_hbm.at[pl.ds(s * RPT_PAD, RPT_PAD)],
                    acc_sh.at[pl.ds(s * RPT_PAD, RPT_PAD)])
    pltpu.sync_copy(dst_hbm.at[w], dst_m)
    plsc.subcore_barrier()

    HW = E_CHUNKS // 2
    for win in range(2):
        kb = win * HW
        pltpu.sync_copy(src_hbm.at[w, pl.ds(kb, HW)], src_m)
        pltpu.async_copy(hs_hbm.at[src_m.at[0]], rows0, sem0)
        pltpu.async_copy(hs_hbm.at[src_m.at[1]], rows1, sem1)

        def body(g, carry):
            k = g * 2
            pltpu.make_async_copy(hs_hbm.at[src_m.at[k]], rows0, sem0).wait()
            pltpu.sync_copy(rows0, acc_sh.at[dst_m.at[kb + k]], add=True)

            @pl.when(k + 2 < HW)
            def _():
                pltpu.async_copy(hs_hbm.at[src_m.at[k + 2]], rows0, sem0)

            pltpu.make_async_copy(hs_hbm.at[src_m.at[k + 1]], rows1, sem1).wait()
            pltpu.sync_copy(rows1, acc_sh.at[dst_m.at[kb + k + 1]], add=True)

            @pl.when(k + 3 < HW)
            def _():
                pltpu.async_copy(hs_hbm.at[src_m.at[k + 3]], rows1, sem1)

            return carry

        lax.fori_loop(0, HW // 2, body, 0)
    plsc.subcore_barrier()
    pltpu.sync_copy(acc_sh.at[pl.ds(s * RPT_PAD, RPT_PAD)],
                    out_hbm.at[c, pl.ds(s * RPT_PAD, RPT_PAD)])


def _prep_body(d_ref, emb_ref, dinv_ref, hs_ref):
    deg = d_ref[0, :, :1] + d_ref[1, :, :1] + 1.0
    dinv = lax.rsqrt(jnp.maximum(deg, 1.0))
    dinv_ref[...] = dinv
    hs_ref[...] = emb_ref[...] * dinv


_BP = 1000


def _prep(degp, emb):
    grid = (N // _BP,)
    return pl.pallas_call(
        _prep_body,
        grid=grid,
        in_specs=[
            pl.BlockSpec((NC, _BP, D), lambda i: (0, i, 0)),
            pl.BlockSpec((_BP, D), lambda i: (i, 0)),
        ],
        out_specs=[
            pl.BlockSpec((_BP, 1), lambda i: (i, 0)),
            pl.BlockSpec((_BP, D), lambda i: (i, 0)),
        ],
        out_shape=[
            jax.ShapeDtypeStruct((N, 1), jnp.float32),
            jax.ShapeDtypeStruct((N, D), jnp.float32),
        ],
    )(degp, emb)


def _layer_body(p_ref, selfrows_ref, dinv_ref, w_ref, out_ref, *, power):
    srows = p_ref[0] + p_ref[1] + selfrows_ref[...]
    x = lax.dot_general(srows, w_ref[...], (((1,), (1,)), ((), ())),
                        preferred_element_type=jnp.float32)
    dinv = dinv_ref[...]
    scale = dinv * dinv if power == 2 else dinv
    out_ref[...] = scale * jnp.maximum(x, 0.0)


def _layer(partials, selfrows, dinv, w, power):
    grid = (N // _BP,)
    return pl.pallas_call(
        functools.partial(_layer_body, power=power),
        grid=grid,
        in_specs=[
            pl.BlockSpec((NC, _BP, D), lambda i: (0, i, 0)),
            pl.BlockSpec((_BP, D), lambda i: (i, 0)),
            pl.BlockSpec((_BP, 1), lambda i: (i, 0)),
            pl.BlockSpec((D, D), lambda i: (0, 0)),
        ],
        out_specs=pl.BlockSpec((_BP, D), lambda i: (i, 0)),
        out_shape=jax.ShapeDtypeStruct((N, D), jnp.float32),
    )(partials, selfrows, dinv, w)


def kernel(emb, W0, W1, edge_index):
    e0 = edge_index[0].astype(jnp.int32)
    e1 = edge_index[1].astype(jnp.int32)
    pad_e = E_PAD - E
    key = jnp.concatenate([e1 * 16384 + e0, jnp.full((pad_e,), N, jnp.int32)])
    key = jnp.sort(key)
    src = (key >> 14).reshape(NW, E_CHUNKS, C)
    dst = (key & 16383).reshape(NW, E_CHUNKS, C)
    ep = jnp.concatenate([e0, e1, jnp.full((EP_PAD - 2 * E,), N, jnp.int32)])
    ep = ep.reshape(NW, EP_CHUNKS, C)
    zeros_tab = jnp.zeros((N_PAD, D), jnp.float32)
    ones_c = jnp.ones((C, D), jnp.float32)

    degp = _deg_kernel(ep, ones_c, zeros_tab)
    dinv, hs0 = _prep(degp, emb)
    p0 = _spmm_kernel(hs0, src, dst, zeros_tab)
    hs1 = _layer(p0, hs0, dinv, W0, 2)
    p1 = _spmm_kernel(hs1, src, dst, zeros_tab)
    return _layer(p1, hs1, dinv, W1, 1)


# speedup vs baseline: 1.2674x; 1.2674x over previous
import functools

import jax
import jax.numpy as jnp
from jax import lax
from jax.experimental import pallas as pl
from jax.experimental.pallas import tpu as pltpu
from jax.experimental.pallas import tpu_sc as plsc

N = 10000
D = 128
E = 320000
NC = 2
NS = 16
NW = NC * NS

C = 128
N_PAD = 10112
RPT_PAD = N_PAD // NS

E_PER_W = 10240
E_PAD = E_PER_W * NW
E_CHUNKS = E_PER_W // C

EP_PER_W = 20480
EP_PAD = EP_PER_W * NW
EP_CHUNKS = EP_PER_W // C

_MESH = plsc.VectorSubcoreMesh(core_axis_name="c", subcore_axis_name="s")


@functools.partial(
    pl.kernel,
    out_type=jax.ShapeDtypeStruct((NC, N_PAD, D), jnp.float32),
    mesh=_MESH,
    scratch_types=[
        pltpu.VMEM((EP_CHUNKS, C), jnp.int32),
        pltpu.VMEM((C, D), jnp.float32),
        pltpu.VMEM_SHARED((N_PAD, D), jnp.float32),
    ],
)
def _deg_kernel(ep_hbm, ones_hbm, zeros_hbm, out_hbm, idx_v, ones_v, acc_sh):
    c = lax.axis_index("c")
    s = lax.axis_index("s")
    w = s * NC + c
    pltpu.sync_copy(zeros_hbm.at[pl.ds(s * RPT_PAD, RPT_PAD)],
                    acc_sh.at[pl.ds(s * RPT_PAD, RPT_PAD)])
    pltpu.sync_copy(ones_hbm, ones_v)
    pltpu.sync_copy(ep_hbm.at[w], idx_v)
    plsc.subcore_barrier()

    def body(k, carry):
        pltpu.sync_copy(ones_v, acc_sh.at[idx_v.at[k]], add=True)
        return carry

    lax.fori_loop(0, EP_CHUNKS, body, 0)
    plsc.subcore_barrier()
    pltpu.sync_copy(acc_sh.at[pl.ds(s * RPT_PAD, RPT_PAD)],
                    out_hbm.at[c, pl.ds(s * RPT_PAD, RPT_PAD)])


@functools.partial(
    pl.kernel,
    out_type=jax.ShapeDtypeStruct((NC, N_PAD, D), jnp.float32),
    mesh=_MESH,
    scratch_types=[
        pltpu.VMEM((E_CHUNKS // 2, C), jnp.int32),
        pltpu.VMEM((E_CHUNKS, C), jnp.int32),
        pltpu.VMEM((C, D), jnp.float32),
        pltpu.VMEM((C, D), jnp.float32),
        pltpu.SemaphoreType.DMA,
        pltpu.SemaphoreType.DMA,
        pltpu.VMEM_SHARED((N_PAD, D), jnp.float32),
    ],
)
def _spmm_kernel(hs_hbm, src_hbm, dst_hbm, zeros_hbm, out_hbm,
                 src_m, dst_m, rows0, rows1, sem0, sem1, acc_sh):
    c = lax.axis_index("c")
    s = lax.axis_index("s")
    w = s * NC + c
    pltpu.sync_copy(zeros_hbm.at[pl.ds(s * RPT_PAD, RPT_PAD)],
                    acc_sh.at[pl.ds(s * RPT_PAD, RPT_PAD)])
    pltpu.sync_copy(dst_hbm.at[w], dst_m)
    plsc.subcore_barrier()

    HW = E_CHUNKS // 2
    for win in range(2):
        kb = win * HW
        pltpu.sync_copy(src_hbm.at[w, pl.ds(kb, HW)], src_m)
        pltpu.async_copy(hs_hbm.at[src_m.at[0]], rows0, sem0)
        pltpu.async_copy(hs_hbm.at[src_m.at[1]], rows1, sem1)

        def body(g, carry):
            k = g * 2
            pltpu.make_async_copy(hs_hbm.at[src_m.at[k]], rows0, sem0).wait()
            pltpu.sync_copy(rows0, acc_sh.at[dst_m.at[kb + k]], add=True)

            @pl.when(k + 2 < HW)
            def _():
                pltpu.async_copy(hs_hbm.at[src_m.at[k + 2]], rows0, sem0)

            pltpu.make_async_copy(hs_hbm.at[src_m.at[k + 1]], rows1, sem1).wait()
            pltpu.sync_copy(rows1, acc_sh.at[dst_m.at[kb + k + 1]], add=True)

            @pl.when(k + 3 < HW)
            def _():
                pltpu.async_copy(hs_hbm.at[src_m.at[k + 3]], rows1, sem1)

            return carry

        lax.fori_loop(0, HW // 2, body, 0)
    plsc.subcore_barrier()
    pltpu.sync_copy(acc_sh.at[pl.ds(s * RPT_PAD, RPT_PAD)],
                    out_hbm.at[c, pl.ds(s * RPT_PAD, RPT_PAD)])


def _prep_body(d_ref, emb_ref, dinv_ref, hs_ref):
    deg = d_ref[0, :, :1] + d_ref[1, :, :1] + 1.0
    dinv = lax.rsqrt(jnp.maximum(deg, 1.0))
    dinv_ref[...] = dinv
    hs_ref[...] = emb_ref[...] * dinv


_BP = 1000


def _prep(degp, emb):
    grid = (N // _BP,)
    return pl.pallas_call(
        _prep_body,
        grid=grid,
        in_specs=[
            pl.BlockSpec((NC, _BP, D), lambda i: (0, i, 0)),
            pl.BlockSpec((_BP, D), lambda i: (i, 0)),
        ],
        out_specs=[
            pl.BlockSpec((_BP, 1), lambda i: (i, 0)),
            pl.BlockSpec((_BP, D), lambda i: (i, 0)),
        ],
        out_shape=[
            jax.ShapeDtypeStruct((N, 1), jnp.float32),
            jax.ShapeDtypeStruct((N, D), jnp.float32),
        ],
    )(degp, emb)


def _layer_body(p_ref, selfrows_ref, dinv_ref, w_ref, out_ref, *, power):
    srows = p_ref[0] + p_ref[1] + selfrows_ref[...]
    x = lax.dot_general(srows, w_ref[...], (((1,), (1,)), ((), ())),
                        preferred_element_type=jnp.float32)
    dinv = dinv_ref[...]
    scale = dinv * dinv if power == 2 else dinv
    out_ref[...] = scale * jnp.maximum(x, 0.0)


def _layer(partials, selfrows, dinv, w, power):
    grid = (N // _BP,)
    return pl.pallas_call(
        functools.partial(_layer_body, power=power),
        grid=grid,
        in_specs=[
            pl.BlockSpec((NC, _BP, D), lambda i: (0, i, 0)),
            pl.BlockSpec((_BP, D), lambda i: (i, 0)),
            pl.BlockSpec((_BP, 1), lambda i: (i, 0)),
            pl.BlockSpec((D, D), lambda i: (0, 0)),
        ],
        out_specs=pl.BlockSpec((_BP, D), lambda i: (i, 0)),
        out_shape=jax.ShapeDtypeStruct((N, D), jnp.float32),
    )(partials, selfrows, dinv, w)


def kernel(emb, W0, W1, edge_index):
    e0 = edge_index[0].astype(jnp.int32)
    e1 = edge_index[1].astype(jnp.int32)
    pad_e = E_PAD - E
    src = jnp.concatenate([e1, jnp.zeros((pad_e,), jnp.int32)])
    src = src.reshape(NW, E_CHUNKS, C)
    dst = jnp.concatenate([e0, jnp.full((pad_e,), N, jnp.int32)])
    dst = dst.reshape(NW, E_CHUNKS, C)
    ep = jnp.concatenate([e0, e1, jnp.full((EP_PAD - 2 * E,), N, jnp.int32)])
    ep = ep.reshape(NW, EP_CHUNKS, C)
    zeros_tab = jnp.zeros((N_PAD, D), jnp.float32)
    ones_c = jnp.ones((C, D), jnp.float32)

    degp = _deg_kernel(ep, ones_c, zeros_tab)
    dinv, hs0 = _prep(degp, emb)
    p0 = _spmm_kernel(hs0, src, dst, zeros_tab)
    hs1 = _layer(p0, hs0, dinv, W0, 2)
    p1 = _spmm_kernel(hs1, src, dst, zeros_tab)
    return _layer(p1, hs1, dinv, W1, 1)
